# baseline (speedup 1.0000x reference)
import functools

import jax
import jax.numpy as jnp
from jax import lax
from jax.experimental import pallas as pl
from jax.experimental.pallas import tpu as pltpu
from jax.experimental.pallas import tpu_sc as plsc

_NC = 2
_NS = 16
_NW = _NC * _NS
_CHUNK = 128


@functools.cache
def _build(V, D, B):
    nchunk = B // (_NW * _CHUNK)
    b_per_w = nchunk * _CHUNK
    mesh = plsc.VectorSubcoreMesh(core_axis_name="c", subcore_axis_name="s")

    @functools.partial(
        pl.kernel,
        mesh=mesh,
        out_type=jax.ShapeDtypeStruct((B, D), jnp.float32),
        scratch_types=[
            pltpu.VMEM((nchunk, _CHUNK), jnp.int32),
            pltpu.VMEM((_CHUNK, D), jnp.float32),
            pltpu.SemaphoreType.DMA,
        ],
        compiler_params=pltpu.CompilerParams(use_tc_tiling_on_sc=False),
    )
    def k(idx_hbm, table_hbm, out_hbm, idx_v, rows_v, sem):
        wid = lax.axis_index("s") * _NC + lax.axis_index("c")
        base = wid * b_per_w
        pltpu.sync_copy(idx_hbm.at[wid], idx_v)

        def body(j, carry):
            pltpu.async_copy(table_hbm.at[idx_v.at[j]], rows_v, sem).wait()
            pltpu.sync_copy(rows_v, out_hbm.at[pl.ds(base + j * _CHUNK, _CHUNK)])
            return carry

        lax.fori_loop(0, nchunk, body, 0)

    return k


def kernel(x, embedding):
    S, T = x.shape
    V, D = embedding.shape
    B = S * T
    xf = x.reshape(_NW, B // (_NW * _CHUNK), _CHUNK)
    out = _build(V, D, B)(xf, embedding)
    return out.reshape(S, T, D)


# speedup vs baseline: 1.1156x; 1.1156x over previous
import functools

import jax
import jax.numpy as jnp
from jax import lax
from jax.experimental import pallas as pl
from jax.experimental.pallas import tpu as pltpu
from jax.experimental.pallas import tpu_sc as plsc

_NC = 2
_NS = 16
_NW = _NC * _NS
_CHUNK = 128
_K = 4


@functools.cache
def _build(V, D, B):
    nchunk = B // (_NW * _CHUNK)
    ngroup = nchunk // _K
    b_per_w = nchunk * _CHUNK
    assert ngroup * _K == nchunk and ngroup >= 4 and ngroup % 2 == 0
    mesh = plsc.VectorSubcoreMesh(core_axis_name="c", subcore_axis_name="s")

    @functools.partial(
        pl.kernel,
        mesh=mesh,
        out_type=jax.ShapeDtypeStruct((B, D), jnp.float32),
        scratch_types=[
            pltpu.VMEM((nchunk, _CHUNK), jnp.int32),
            pltpu.VMEM((_K, _CHUNK, D), jnp.float32),
            pltpu.VMEM((_K, _CHUNK, D), jnp.float32),
            pltpu.SemaphoreType.DMA,
            pltpu.SemaphoreType.DMA,
            pltpu.SemaphoreType.DMA,
            pltpu.SemaphoreType.DMA,
        ],
        compiler_params=pltpu.CompilerParams(use_tc_tiling_on_sc=False),
    )
    def k(idx_hbm, table_hbm, out_hbm, idx_v, rows0, rows1,
          gsem0, gsem1, osem0, osem1):
        wid = lax.axis_index("s") * _NC + lax.axis_index("c")
        base = wid * b_per_w
        pltpu.sync_copy(idx_hbm.at[wid], idx_v)

        def fire_gather(rows, gsem, g):
            for b in range(_K):
                j = g * _K + b
                pltpu.async_copy(table_hbm.at[idx_v.at[j]], rows.at[b], gsem)

        def drain_gather(rows, gsem, g):
            for b in range(_K):
                j = g * _K + b
                pltpu.make_async_copy(
                    table_hbm.at[idx_v.at[j]], rows.at[b], gsem).wait()

        def fire_out(rows, osem, g):
            for b in range(_K):
                j = g * _K + b
                pltpu.async_copy(
                    rows.at[b], out_hbm.at[pl.ds(base + j * _CHUNK, _CHUNK)],
                    osem)

        def drain_out(rows, osem, g):
            for b in range(_K):
                j = g * _K + b
                pltpu.make_async_copy(
                    rows.at[b], out_hbm.at[pl.ds(base + j * _CHUNK, _CHUNK)],
                    osem).wait()

        set0 = (rows0, gsem0, osem0)
        set1 = (rows1, gsem1, osem1)

        def step(g, cur, oth, first=False, fire_next=True):
            rows_c, gsem_c, osem_c = cur
            rows_o, gsem_o, osem_o = oth
            if not first:
                drain_out(rows_o, osem_o, g - 1)
            if fire_next:
                fire_gather(rows_o, gsem_o, g + 1)
            drain_gather(rows_c, gsem_c, g)
            fire_out(rows_c, osem_c, g)

        fire_gather(rows0, gsem0, 0)
        step(0, set0, set1, first=True)
        step(1, set1, set0)

        def body(m, carry):
            step(2 * m, set0, set1)
            step(2 * m + 1, set1, set0)
            return carry

        lax.fori_loop(1, ngroup // 2 - 1, body, 0)

        step(ngroup - 2, set0, set1)
        step(ngroup - 1, set1, set0, fire_next=False)
        drain_out(rows1, osem1, ngroup - 1)

    return k


def kernel(x, embedding):
    S, T = x.shape
    V, D = embedding.shape
    B = S * T
    xf = x.reshape(_NW, B // (_NW * _CHUNK), _CHUNK)
    out = _build(V, D, B)(xf, embedding)
    return out.reshape(S, T, D)
